# baseline (speedup 1.0000x reference)
import jax
import jax.numpy as jnp
from jax import lax
from jax.experimental import pallas as pl
from jax.experimental.pallas import tpu as pltpu
from jax.experimental.pallas import tpu_sc as plsc

STYLE_N = 1000
FRAME_N = 200
DIM = 128
BATCH = 4096

NUM_CORES = 2
NUM_SUBCORES = 16
LANES = 16
NW = NUM_CORES * NUM_SUBCORES
B_PER_W = BATCH // NW


CHUNK = 32
NCHUNK = B_PER_W // CHUNK


def _gather_body(style_hbm, frame_hbm, table_hbm, out_hbm,
                 sidx_v, fidx_v, flat_v, rows_v, gsem, wsem):
    wid = lax.axis_index("s") * NUM_CORES + lax.axis_index("c")
    base = wid * B_PER_W

    pltpu.sync_copy(style_hbm.at[pl.ds(base, B_PER_W)], sidx_v)
    pltpu.sync_copy(frame_hbm.at[pl.ds(base, B_PER_W)], fidx_v)

    for i in range(B_PER_W // LANES):
        s = sidx_v[pl.ds(i * LANES, LANES)]
        f = fidx_v[pl.ds(i * LANES, LANES)]
        flat_v[pl.ds(i * LANES, LANES)] = s * FRAME_N + f

    gathers = []
    for c in range(NCHUNK):
        gathers.append(pltpu.async_copy(
            table_hbm.at[flat_v.at[pl.ds(c * CHUNK, CHUNK)]],
            rows_v.at[pl.ds(c * CHUNK, CHUNK)], gsem))
    writes = []
    for c in range(NCHUNK):
        gathers[c].wait()
        writes.append(pltpu.async_copy(
            rows_v.at[pl.ds(c * CHUNK, CHUNK)],
            out_hbm.at[pl.ds(base + c * CHUNK, CHUNK)], wsem))
    for w in writes:
        w.wait()


@jax.jit
def _sc_gather(style_ids, frame_ids, table):
    mesh = plsc.VectorSubcoreMesh(core_axis_name="c", subcore_axis_name="s")
    return pl.kernel(
        _gather_body,
        mesh=mesh,
        out_type=jax.ShapeDtypeStruct((BATCH, DIM), jnp.float32),
        scratch_types=[
            pltpu.VMEM((B_PER_W,), jnp.int32),
            pltpu.VMEM((B_PER_W,), jnp.int32),
            pltpu.VMEM((B_PER_W,), jnp.int32),
            pltpu.VMEM((B_PER_W, DIM), jnp.float32),
            pltpu.SemaphoreType.DMA,
            pltpu.SemaphoreType.DMA,
        ],
    )(style_ids, frame_ids, table)


def kernel(style_ids, frame_ids, latents, latents_mu):
    del latents_mu
    table = latents.reshape(STYLE_N * FRAME_N, DIM)
    return _sc_gather(style_ids.astype(jnp.int32), frame_ids.astype(jnp.int32),
                      table)


# speedup vs baseline: 1.0327x; 1.0327x over previous
import jax
import jax.numpy as jnp
from jax import lax
from jax.experimental import pallas as pl
from jax.experimental.pallas import tpu as pltpu
from jax.experimental.pallas import tpu_sc as plsc

STYLE_N = 1000
FRAME_N = 200
DIM = 128
BATCH = 4096

NUM_CORES = 2
NUM_SUBCORES = 16
LANES = 16
NW = NUM_CORES * NUM_SUBCORES
B_PER_W = BATCH // NW


CHUNK = 64
NCHUNK = B_PER_W // CHUNK


def _gather_body(style_hbm, frame_hbm, table_hbm, out_hbm,
                 sidx_v, fidx_v, flat_v, rows_v, gsem, wsem):
    wid = lax.axis_index("s") * NUM_CORES + lax.axis_index("c")
    base = wid * B_PER_W

    c1 = pltpu.async_copy(style_hbm.at[pl.ds(base, B_PER_W)], sidx_v, wsem)
    c2 = pltpu.async_copy(frame_hbm.at[pl.ds(base, B_PER_W)], fidx_v, wsem)
    c1.wait()
    c2.wait()

    for i in range(B_PER_W // LANES):
        s = sidx_v[pl.ds(i * LANES, LANES)]
        f = fidx_v[pl.ds(i * LANES, LANES)]
        flat_v[pl.ds(i * LANES, LANES)] = s * FRAME_N + f

    gathers = []
    for c in range(NCHUNK):
        gathers.append(pltpu.async_copy(
            table_hbm.at[flat_v.at[pl.ds(c * CHUNK, CHUNK)]],
            rows_v.at[pl.ds(c * CHUNK, CHUNK)], gsem))
    writes = []
    for c in range(NCHUNK):
        gathers[c].wait()
        writes.append(pltpu.async_copy(
            rows_v.at[pl.ds(c * CHUNK, CHUNK)],
            out_hbm.at[pl.ds(base + c * CHUNK, CHUNK)], wsem))
    for w in writes:
        w.wait()


@jax.jit
def _sc_gather(style_ids, frame_ids, table):
    mesh = plsc.VectorSubcoreMesh(core_axis_name="c", subcore_axis_name="s")
    return pl.kernel(
        _gather_body,
        mesh=mesh,
        out_type=jax.ShapeDtypeStruct((BATCH, DIM), jnp.float32),
        scratch_types=[
            pltpu.VMEM((B_PER_W,), jnp.int32),
            pltpu.VMEM((B_PER_W,), jnp.int32),
            pltpu.VMEM((B_PER_W,), jnp.int32),
            pltpu.VMEM((B_PER_W, DIM), jnp.float32),
            pltpu.SemaphoreType.DMA,
            pltpu.SemaphoreType.DMA,
        ],
    )(style_ids, frame_ids, table)


def kernel(style_ids, frame_ids, latents, latents_mu):
    del latents_mu
    table = latents.reshape(STYLE_N * FRAME_N, DIM)
    return _sc_gather(style_ids.astype(jnp.int32), frame_ids.astype(jnp.int32),
                      table)
